# baseline (speedup 1.0000x reference)
import functools

import jax
import jax.numpy as jnp
from jax import lax
from jax.experimental import pallas as pl
from jax.experimental.pallas import tpu as pltpu
from jax.experimental.pallas import tpu_sc as plsc

N_REAL = 10000
N_PAD = 10240
DUMMY = 10000
D = 128
E_REAL = 320000
NW = 32
EC = 64
ROWS_W = 160
E_ROWS = NW * ROWS_W
E_PAD = E_ROWS * EC
MSG_PAD = E_PAD + 4 * EC
SLICE = N_PAD // 16
WB = SLICE // EC
BLK = 1024
GRID = N_PAD // BLK

_MESH = dict(core_axis_name="c", subcore_axis_name="s")


def _deg_partials(dst2):
    @functools.partial(
        pl.kernel,
        out_type=jax.ShapeDtypeStruct((2, N_PAD), jnp.float32),
        mesh=plsc.VectorSubcoreMesh(**_MESH),
        scratch_types=[
            pltpu.VMEM((ROWS_W, EC), jnp.int32),
            pltpu.VMEM((EC,), jnp.float32),
            pltpu.VMEM((SLICE,), jnp.float32),
            pltpu.VMEM_SHARED((N_PAD,), jnp.float32),
        ],
    )
    def k(dst_hbm, out_hbm, dst_v, ones_v, zbuf, cnt):
        c = lax.axis_index("c")
        s = lax.axis_index("s")
        wid = s * 2 + c

        def fill_ones(i, _):
            ones_v[pl.ds(i * 16, 16)] = jnp.ones((16,), jnp.float32)
            return 0

        lax.fori_loop(0, EC // 16, fill_ones, 0)

        def fill_zeros(i, _):
            zbuf[pl.ds(i * 16, 16)] = jnp.zeros((16,), jnp.float32)
            return 0

        lax.fori_loop(0, SLICE // 16, fill_zeros, 0)
        pltpu.sync_copy(zbuf, cnt.at[pl.ds(s * SLICE, SLICE)])
        plsc.subcore_barrier()

        pltpu.sync_copy(dst_hbm.at[pl.ds(wid * ROWS_W, ROWS_W)], dst_v)

        def body(j, _):
            pltpu.sync_copy(ones_v, cnt.at[dst_v.at[j]], add=True)
            return 0

        lax.fori_loop(0, ROWS_W, body, 0)
        plsc.subcore_barrier()
        pltpu.sync_copy(cnt.at[pl.ds(s * SLICE, SLICE)],
                        out_hbm.at[c, pl.ds(s * SLICE, SLICE)])

    return k(dst2)


def _spmm_partials(g, e2):
    @functools.partial(
        pl.kernel,
        out_type=[
            jax.ShapeDtypeStruct((2, N_PAD, D), jnp.float32),
            jax.ShapeDtypeStruct((MSG_PAD, D), jnp.float32),
        ],
        mesh=plsc.VectorSubcoreMesh(**_MESH),
        scratch_types=[
            pltpu.VMEM((2, EC), jnp.int32),
            pltpu.VMEM((2, EC), jnp.int32),
            pltpu.VMEM((2, EC), jnp.int32),
            pltpu.VMEM((2, EC), jnp.int32),
            pltpu.VMEM((2, EC), jnp.int32),
            pltpu.VMEM((2, EC), jnp.int32),
            pltpu.VMEM((2, EC), jnp.int32),
            pltpu.VMEM((2, EC), jnp.int32),
            pltpu.VMEM((EC, D), jnp.float32),
            pltpu.VMEM((EC, D), jnp.float32),
            pltpu.VMEM((EC, D), jnp.float32),
            pltpu.VMEM((EC, D), jnp.float32),
            pltpu.VMEM_SHARED((N_PAD, D), jnp.float32),
            pltpu.SemaphoreType.DMA,
            pltpu.SemaphoreType.DMA,
            pltpu.SemaphoreType.DMA,
            pltpu.SemaphoreType.DMA,
            pltpu.SemaphoreType.DMA,
            pltpu.SemaphoreType.DMA,
            pltpu.SemaphoreType.DMA,
            pltpu.SemaphoreType.DMA,
            pltpu.SemaphoreType.DMA,
            pltpu.SemaphoreType.DMA,
            pltpu.SemaphoreType.DMA,
            pltpu.SemaphoreType.DMA,
        ],
    )
    def k(g_hbm, e_hbm, out_hbm, msg_hbm,
          i0, i1, i2, i3, i4, i5, i6, i7, b0, b1, b2, b3, shp,
          si0, si1, si2, si3, si4, si5, si6, si7, sg0, sg1, sg2, sg3):
        c = lax.axis_index("c")
        s = lax.axis_index("s")
        wid = s * 2 + c
        base = wid * ROWS_W
        ibufs = [i0, i1, i2, i3, i4, i5, i6, i7]
        isems = [si0, si1, si2, si3, si4, si5, si6, si7]
        gbufs = [b0, b1, b2, b3]
        gsems = [sg0, sg1, sg2, sg3]

        def istart(row, ib, sem):
            pltpu.async_copy(e_hbm.at[row], ib, sem)

        def iwait(ib, sem):
            pltpu.make_async_copy(e_hbm.at[0], ib, sem).wait()

        def gstart(ib, gb, sem):
            pltpu.async_copy(shp.at[ib.at[0]], gb, sem)

        def gwait(gb, sem):
            pltpu.make_async_copy(shp.at[i0.at[0]], gb, sem).wait()

        def mstart(row, mb, sem):
            pltpu.async_copy(msg_hbm.at[pl.ds(row * EC, EC)], mb, sem)

        def mwait(mb, sem):
            pltpu.make_async_copy(msg_hbm.at[pl.ds(0, EC)], mb, sem).wait()

        for r in range(8):
            istart(base + r, ibufs[r], isems[r])
        pltpu.sync_copy(g_hbm.at[pl.ds(s * SLICE, SLICE)],
                        shp.at[pl.ds(s * SLICE, SLICE)])
        plsc.subcore_barrier()
        for b in range(4):
            iwait(ibufs[b], isems[b])
            gstart(ibufs[b], gbufs[b], gsems[b])

        def body_a(jj, _):
            j0 = jj * 8
            for t in range(8):
                bi, ib4 = t % 4, (t + 4) % 8
                gwait(gbufs[bi], gsems[bi])
                pltpu.sync_copy(gbufs[bi],
                                msg_hbm.at[pl.ds((base + j0 + t) * EC, EC)])
                istart(base + j0 + t + 8, ibufs[t], isems[t])
                iwait(ibufs[ib4], isems[ib4])
                gstart(ibufs[ib4], gbufs[bi], gsems[bi])
            return 0

        lax.fori_loop(0, ROWS_W // 8, body_a, 0)
        for b in range(4):
            gwait(gbufs[b], gsems[b])
        for r in range(4, 8):
            iwait(ibufs[r], isems[r])
        plsc.subcore_barrier()

        for r in range(8):
            istart(base + r, ibufs[r], isems[r])

        def fill_zeros(i, _):
            r = i // (D // 16)
            col = (i % (D // 16)) * 16
            b0[r, pl.ds(col, 16)] = jnp.zeros((16,), jnp.float32)
            return 0

        lax.fori_loop(0, EC * (D // 16), fill_zeros, 0)
        for j in range(WB):
            pltpu.sync_copy(b0, shp.at[pl.ds(s * SLICE + j * EC, EC)])
        for b in range(4):
            mstart(base + b, gbufs[b], gsems[b])
        plsc.subcore_barrier()

        def body_b(jj, _):
            j0 = jj * 8
            for t in range(8):
                bi = t % 4
                mwait(gbufs[bi], gsems[bi])
                iwait(ibufs[t], isems[t])
                pltpu.sync_copy(gbufs[bi], shp.at[ibufs[t].at[1]], add=True)
                istart(base + j0 + t + 8, ibufs[t], isems[t])
                mstart(base + j0 + t + 4, gbufs[bi], gsems[bi])
            return 0

        lax.fori_loop(0, ROWS_W // 8, body_b, 0)
        for b in range(4):
            mwait(gbufs[b], gsems[b])
        for r in range(8):
            iwait(ibufs[r], isems[r])
        plsc.subcore_barrier()
        for j in range(WB):
            pltpu.sync_copy(shp.at[pl.ds(s * SLICE + j * EC, EC)],
                            out_hbm.at[c, pl.ds(s * SLICE + j * EC, EC)])

    return k(g, e2)[0]


def _tcmm(x_pad, W1):
    def body(x_ref, w_ref, o_ref):
        o_ref[...] = jnp.dot(x_ref[...], w_ref[...],
                             preferred_element_type=jnp.float32)

    return pl.pallas_call(
        body,
        grid=(GRID,),
        in_specs=[
            pl.BlockSpec((BLK, D), lambda i: (i, 0)),
            pl.BlockSpec((D, D), lambda i: (0, 0)),
        ],
        out_specs=pl.BlockSpec((BLK, D), lambda i: (i, 0)),
        out_shape=jax.ShapeDtypeStruct((N_PAD, D), jnp.float32),
    )(x_pad, W1)


def _tc1(degp, xW):
    def body(deg_ref, xw_ref, g_ref, dinv_ref):
        i = pl.program_id(0)
        deg = deg_ref[0, pl.ds(i * BLK, BLK)] + deg_ref[1, pl.ds(i * BLK, BLK)] + 1.0
        dinv = lax.rsqrt(deg)
        dinv_ref[...] = dinv[:, None]
        g_ref[...] = dinv[:, None] * xw_ref[...]

    return pl.pallas_call(
        body,
        grid=(GRID,),
        in_specs=[
            pl.BlockSpec((2, N_PAD), lambda i: (0, 0)),
            pl.BlockSpec((BLK, D), lambda i: (i, 0)),
        ],
        out_specs=[
            pl.BlockSpec((BLK, D), lambda i: (i, 0)),
            pl.BlockSpec((BLK, 1), lambda i: (i, 0)),
        ],
        out_shape=[
            jax.ShapeDtypeStruct((N_PAD, D), jnp.float32),
            jax.ShapeDtypeStruct((N_PAD, 1), jnp.float32),
        ],
    )(degp, xW)


def _tc2(P, g1, dinv, b1, W2):
    def body(p_ref, g_ref, dinv_ref, b_ref, w_ref, o_ref):
        dinv_c = dinv_ref[...]
        h = jnp.maximum(dinv_c * (p_ref[0] + p_ref[1] + g_ref[...]) + b_ref[...], 0.0)
        o_ref[...] = dinv_c * jnp.dot(h, w_ref[...], preferred_element_type=jnp.float32)

    return pl.pallas_call(
        body,
        grid=(GRID,),
        in_specs=[
            pl.BlockSpec((2, BLK, D), lambda i: (0, i, 0)),
            pl.BlockSpec((BLK, D), lambda i: (i, 0)),
            pl.BlockSpec((BLK, 1), lambda i: (i, 0)),
            pl.BlockSpec((1, D), lambda i: (0, 0)),
            pl.BlockSpec((D, D), lambda i: (0, 0)),
        ],
        out_specs=pl.BlockSpec((BLK, D), lambda i: (i, 0)),
        out_shape=jax.ShapeDtypeStruct((N_PAD, D), jnp.float32),
    )(P, g1, dinv, b1, W2)


def _tc3(P, g2, dinv, b2):
    def body(p_ref, g_ref, dinv_ref, b_ref, o_ref):
        o_ref[...] = dinv_ref[...] * (p_ref[0] + p_ref[1] + g_ref[...]) + b_ref[...]

    return pl.pallas_call(
        body,
        grid=(GRID,),
        in_specs=[
            pl.BlockSpec((2, BLK, D), lambda i: (0, i, 0)),
            pl.BlockSpec((BLK, D), lambda i: (i, 0)),
            pl.BlockSpec((BLK, 1), lambda i: (i, 0)),
            pl.BlockSpec((1, D), lambda i: (0, 0)),
        ],
        out_specs=pl.BlockSpec((BLK, D), lambda i: (i, 0)),
        out_shape=jax.ShapeDtypeStruct((N_PAD, D), jnp.float32),
    )(P, g2, dinv, b2)


def kernel(x, edge_index, W1, b1, W2, b2):
    src = edge_index[0].astype(jnp.int32)
    dst = edge_index[1].astype(jnp.int32)
    pad = jnp.full((E_PAD - E_REAL,), DUMMY, jnp.int32)
    srcp = jnp.concatenate([src, pad]).reshape(E_ROWS, 1, EC)
    dstp = jnp.concatenate([dst, pad]).reshape(E_ROWS, 1, EC)
    tail = jnp.full((8, 2, EC), DUMMY, jnp.int32)
    e2 = jnp.concatenate(
        [jnp.concatenate([srcp, dstp], axis=1), tail], axis=0)
    dst2 = dstp.reshape(E_ROWS, EC)
    x_pad = jnp.zeros((N_PAD, D), jnp.float32).at[:N_REAL].set(x)

    degp = _deg_partials(dst2)
    xW = _tcmm(x_pad, W1)
    g1, dinv = _tc1(degp, xW)
    P1 = _spmm_partials(g1, e2)
    g2 = _tc2(P1, g1, dinv, b1.reshape(1, D), W2)
    P2 = _spmm_partials(g2, e2)
    z = _tc3(P2, g2, dinv, b2.reshape(1, D))
    return z[:N_REAL]


# speedup vs baseline: 1.0173x; 1.0173x over previous
import functools

import jax
import jax.numpy as jnp
from jax import lax
from jax.experimental import pallas as pl
from jax.experimental.pallas import tpu as pltpu
from jax.experimental.pallas import tpu_sc as plsc

N_REAL = 10000
N_PAD = 10240
DUMMY = 10000
D = 128
E_REAL = 320000
NW = 32
EC = 64
ROWS_W = 160
E_ROWS = NW * ROWS_W
E_PAD = E_ROWS * EC
MSG_PAD = E_PAD + 4 * EC
SLICE = N_PAD // 16
WB = SLICE // EC
BLK = 1024
GRID = N_PAD // BLK

_MESH = dict(core_axis_name="c", subcore_axis_name="s")


def _deg_partials(dst2):
    @functools.partial(
        pl.kernel,
        out_type=jax.ShapeDtypeStruct((2, N_PAD), jnp.float32),
        mesh=plsc.VectorSubcoreMesh(**_MESH),
        scratch_types=[
            pltpu.VMEM((ROWS_W, EC), jnp.int32),
            pltpu.VMEM((EC,), jnp.float32),
            pltpu.VMEM((SLICE,), jnp.float32),
            pltpu.VMEM_SHARED((N_PAD,), jnp.float32),
        ],
    )
    def k(dst_hbm, out_hbm, dst_v, ones_v, zbuf, cnt):
        c = lax.axis_index("c")
        s = lax.axis_index("s")
        wid = s * 2 + c

        def fill_ones(i, _):
            ones_v[pl.ds(i * 16, 16)] = jnp.ones((16,), jnp.float32)
            return 0

        lax.fori_loop(0, EC // 16, fill_ones, 0)

        def fill_zeros(i, _):
            zbuf[pl.ds(i * 16, 16)] = jnp.zeros((16,), jnp.float32)
            return 0

        lax.fori_loop(0, SLICE // 16, fill_zeros, 0)
        pltpu.sync_copy(zbuf, cnt.at[pl.ds(s * SLICE, SLICE)])
        plsc.subcore_barrier()

        pltpu.sync_copy(dst_hbm.at[pl.ds(wid * ROWS_W, ROWS_W)], dst_v)

        def body(j, _):
            pltpu.sync_copy(ones_v, cnt.at[dst_v.at[j]], add=True)
            return 0

        lax.fori_loop(0, ROWS_W, body, 0)
        plsc.subcore_barrier()
        pltpu.sync_copy(cnt.at[pl.ds(s * SLICE, SLICE)],
                        out_hbm.at[c, pl.ds(s * SLICE, SLICE)])

    return k(dst2)


def _spmm_partials(g, e2):
    @functools.partial(
        pl.kernel,
        out_type=[
            jax.ShapeDtypeStruct((2, N_PAD, D), jnp.float32),
            jax.ShapeDtypeStruct((MSG_PAD, D), jnp.float32),
        ],
        mesh=plsc.VectorSubcoreMesh(**_MESH),
        scratch_types=[
            pltpu.VMEM((2, EC), jnp.int32),
            pltpu.VMEM((2, EC), jnp.int32),
            pltpu.VMEM((2, EC), jnp.int32),
            pltpu.VMEM((2, EC), jnp.int32),
            pltpu.VMEM((2, EC), jnp.int32),
            pltpu.VMEM((2, EC), jnp.int32),
            pltpu.VMEM((2, EC), jnp.int32),
            pltpu.VMEM((2, EC), jnp.int32),
            pltpu.VMEM((EC, D), jnp.float32),
            pltpu.VMEM((EC, D), jnp.float32),
            pltpu.VMEM((EC, D), jnp.float32),
            pltpu.VMEM((EC, D), jnp.float32),
            pltpu.VMEM_SHARED((N_PAD, D), jnp.float32),
            pltpu.SemaphoreType.DMA,
            pltpu.SemaphoreType.DMA,
            pltpu.SemaphoreType.DMA,
            pltpu.SemaphoreType.DMA,
            pltpu.SemaphoreType.DMA,
            pltpu.SemaphoreType.DMA,
            pltpu.SemaphoreType.DMA,
            pltpu.SemaphoreType.DMA,
            pltpu.SemaphoreType.DMA,
            pltpu.SemaphoreType.DMA,
            pltpu.SemaphoreType.DMA,
            pltpu.SemaphoreType.DMA,
        ],
    )
    def k(g_hbm, e_hbm, out_hbm, msg_hbm,
          i0, i1, i2, i3, i4, i5, i6, i7, b0, b1, b2, b3, shp,
          si0, si1, si2, si3, si4, si5, si6, si7, sg0, sg1, sg2, sg3):
        c = lax.axis_index("c")
        s = lax.axis_index("s")
        wid = s * 2 + c
        base = wid * ROWS_W
        ibufs = [i0, i1, i2, i3, i4, i5, i6, i7]
        isems = [si0, si1, si2, si3, si4, si5, si6, si7]
        gbufs = [b0, b1, b2, b3]
        gsems = [sg0, sg1, sg2, sg3]

        def istart(row, ib, sem):
            pltpu.async_copy(e_hbm.at[row], ib, sem)

        def iwait(ib, sem):
            pltpu.make_async_copy(e_hbm.at[0], ib, sem).wait()

        def gstart(ib, gb, sem):
            pltpu.async_copy(shp.at[ib.at[0]], gb, sem)

        def gwait(gb, sem):
            pltpu.make_async_copy(shp.at[i0.at[0]], gb, sem).wait()

        def mstart(row, mb, sem):
            pltpu.async_copy(msg_hbm.at[pl.ds(row * EC, EC)], mb, sem)

        def mwait(mb, sem):
            pltpu.make_async_copy(msg_hbm.at[pl.ds(0, EC)], mb, sem).wait()

        for r in range(8):
            istart(base + r, ibufs[r], isems[r])
        pltpu.sync_copy(g_hbm.at[pl.ds(s * SLICE, SLICE)],
                        shp.at[pl.ds(s * SLICE, SLICE)])
        plsc.subcore_barrier()
        for b in range(4):
            iwait(ibufs[b], isems[b])
            gstart(ibufs[b], gbufs[b], gsems[b])

        def body_a(jj, _):
            j0 = jj * 8
            for t in range(8):
                bi, ib4 = t % 4, (t + 4) % 8
                gwait(gbufs[bi], gsems[bi])
                pltpu.sync_copy(gbufs[bi],
                                msg_hbm.at[pl.ds((base + j0 + t) * EC, EC)])
                istart(base + j0 + t + 8, ibufs[t], isems[t])
                iwait(ibufs[ib4], isems[ib4])
                gstart(ibufs[ib4], gbufs[bi], gsems[bi])
            return 0

        lax.fori_loop(0, ROWS_W // 8, body_a, 0)
        for b in range(4):
            gwait(gbufs[b], gsems[b])
        for r in range(4, 8):
            iwait(ibufs[r], isems[r])
        plsc.subcore_barrier()

        for r in range(8):
            istart(base + r, ibufs[r], isems[r])

        def fill_zeros(i, _):
            r = i // (D // 16)
            col = (i % (D // 16)) * 16
            b0[r, pl.ds(col, 16)] = jnp.zeros((16,), jnp.float32)
            return 0

        lax.fori_loop(0, EC * (D // 16), fill_zeros, 0)
        for j in range(WB):
            pltpu.sync_copy(b0, shp.at[pl.ds(s * SLICE + j * EC, EC)])
        for b in range(4):
            mstart(base + b, gbufs[b], gsems[b])
        plsc.subcore_barrier()

        def body_b(jj, _):
            j0 = jj * 8
            for t in range(8):
                bi = t % 4
                mwait(gbufs[bi], gsems[bi])
                iwait(ibufs[t], isems[t])
                pltpu.sync_copy(gbufs[bi], shp.at[ibufs[t].at[1]], add=True)
                istart(base + j0 + t + 8, ibufs[t], isems[t])
                mstart(base + j0 + t + 4, gbufs[bi], gsems[bi])
            return 0

        lax.fori_loop(0, ROWS_W // 8, body_b, 0)
        for b in range(4):
            mwait(gbufs[b], gsems[b])
        for r in range(8):
            iwait(ibufs[r], isems[r])
        plsc.subcore_barrier()
        for j in range(WB):
            pltpu.sync_copy(shp.at[pl.ds(s * SLICE + j * EC, EC)],
                            out_hbm.at[c, pl.ds(s * SLICE + j * EC, EC)])

    return k(g, e2)[0]


def _tc1(degp, x_pad, W1):
    def body(deg_ref, x_ref, w_ref, g_ref, dinv_ref):
        i = pl.program_id(0)
        deg = deg_ref[0, pl.ds(i * BLK, BLK)] + deg_ref[1, pl.ds(i * BLK, BLK)] + 1.0
        dinv = lax.rsqrt(deg)
        dinv_ref[...] = dinv[:, None]
        g_ref[...] = dinv[:, None] * jnp.dot(
            x_ref[...], w_ref[...], preferred_element_type=jnp.float32)

    return pl.pallas_call(
        body,
        grid=(GRID,),
        in_specs=[
            pl.BlockSpec((2, N_PAD), lambda i: (0, 0)),
            pl.BlockSpec((BLK, D), lambda i: (i, 0)),
            pl.BlockSpec((D, D), lambda i: (0, 0)),
        ],
        out_specs=[
            pl.BlockSpec((BLK, D), lambda i: (i, 0)),
            pl.BlockSpec((BLK, 1), lambda i: (i, 0)),
        ],
        out_shape=[
            jax.ShapeDtypeStruct((N_PAD, D), jnp.float32),
            jax.ShapeDtypeStruct((N_PAD, 1), jnp.float32),
        ],
    )(degp, x_pad, W1)


def _tc2(P, g1, dinv, b1, W2):
    def body(p_ref, g_ref, dinv_ref, b_ref, w_ref, o_ref):
        dinv_c = dinv_ref[...]
        h = jnp.maximum(dinv_c * (p_ref[0] + p_ref[1] + g_ref[...]) + b_ref[...], 0.0)
        o_ref[...] = dinv_c * jnp.dot(h, w_ref[...], preferred_element_type=jnp.float32)

    return pl.pallas_call(
        body,
        grid=(GRID,),
        in_specs=[
            pl.BlockSpec((2, BLK, D), lambda i: (0, i, 0)),
            pl.BlockSpec((BLK, D), lambda i: (i, 0)),
            pl.BlockSpec((BLK, 1), lambda i: (i, 0)),
            pl.BlockSpec((1, D), lambda i: (0, 0)),
            pl.BlockSpec((D, D), lambda i: (0, 0)),
        ],
        out_specs=pl.BlockSpec((BLK, D), lambda i: (i, 0)),
        out_shape=jax.ShapeDtypeStruct((N_PAD, D), jnp.float32),
    )(P, g1, dinv, b1, W2)


def _tc3(P, g2, dinv, b2):
    def body(p_ref, g_ref, dinv_ref, b_ref, o_ref):
        o_ref[...] = dinv_ref[...] * (p_ref[0] + p_ref[1] + g_ref[...]) + b_ref[...]

    return pl.pallas_call(
        body,
        grid=(GRID,),
        in_specs=[
            pl.BlockSpec((2, BLK, D), lambda i: (0, i, 0)),
            pl.BlockSpec((BLK, D), lambda i: (i, 0)),
            pl.BlockSpec((BLK, 1), lambda i: (i, 0)),
            pl.BlockSpec((1, D), lambda i: (0, 0)),
        ],
        out_specs=pl.BlockSpec((BLK, D), lambda i: (i, 0)),
        out_shape=jax.ShapeDtypeStruct((N_PAD, D), jnp.float32),
    )(P, g2, dinv, b2)


def kernel(x, edge_index, W1, b1, W2, b2):
    src = edge_index[0].astype(jnp.int32)
    dst = edge_index[1].astype(jnp.int32)
    pad = jnp.full((E_PAD - E_REAL,), DUMMY, jnp.int32)
    srcp = jnp.concatenate([src, pad]).reshape(E_ROWS, 1, EC)
    dstp = jnp.concatenate([dst, pad]).reshape(E_ROWS, 1, EC)
    tail = jnp.full((8, 2, EC), DUMMY, jnp.int32)
    e2 = jnp.concatenate(
        [jnp.concatenate([srcp, dstp], axis=1), tail], axis=0)
    dst2 = dstp.reshape(E_ROWS, EC)
    x_pad = jnp.zeros((N_PAD, D), jnp.float32).at[:N_REAL].set(x)

    degp = _deg_partials(dst2)
    g1, dinv = _tc1(degp, x_pad, W1)
    P1 = _spmm_partials(g1, e2)
    g2 = _tc2(P1, g1, dinv, b1.reshape(1, D), W2)
    P2 = _spmm_partials(g2, e2)
    z = _tc3(P2, g2, dinv, b2.reshape(1, D))
    return z[:N_REAL]
